# baseline (speedup 1.0000x reference)
import functools

import jax
import jax.numpy as jnp
from jax import lax
from jax.experimental import pallas as pl
from jax.experimental.pallas import tpu as pltpu
from jax.experimental.pallas import tpu_sc as plsc

NC = 2
NS = 16
NW = NC * NS
W = 128
NBUF = 4
TUNROLL = 8


@functools.lru_cache(maxsize=None)
def _make_lookup(S0, S1, D):
    B = S0 * S1
    NA = S0 // W
    b_per_w = B // NW
    nblk = b_per_w // W
    ngroup = nblk // NBUF
    assert b_per_w * NW == B and W * nblk == b_per_w and NBUF * ngroup == nblk
    assert D == 64 and S0 % W == 0

    mesh = plsc.VectorSubcoreMesh(core_axis_name="c", subcore_axis_name="s")

    @functools.partial(
        pl.kernel,
        mesh=mesh,
        compiler_params=pltpu.CompilerParams(
            use_tc_tiling_on_sc=False, needs_layout_passes=False),
        out_type=(
            jax.ShapeDtypeStruct((S1, 8, NA, 8, W), jnp.float32),
            jax.ShapeDtypeStruct((S1, 8, NA, 8, W), jnp.float32),
        ),
        scratch_types=(
            [pltpu.VMEM((b_per_w,), jnp.int32)] * 2
            + [pltpu.VMEM((W, D), jnp.float32)] * NBUF
            + [pltpu.VMEM((8, 8, W), jnp.float32)] * NBUF
            + [pltpu.SemaphoreType.DMA] * (2 * NBUF)
        ),
    )
    def lookup(tok_idx_hbm, pos_idx_hbm, tok_tab, pos_tab, tok_out, pos_out,
               tok_idx_v, pos_idx_v, *scratch):
        rows = scratch[:NBUF]
        rowsT = scratch[NBUF:2 * NBUF]
        gsems = scratch[2 * NBUF:3 * NBUF]
        osems = scratch[3 * NBUF:]

        wid = lax.axis_index("s") * NC + lax.axis_index("c")
        ibase = pl.multiple_of(wid * b_per_w, 8)
        gbase = wid * nblk

        pltpu.sync_copy(tok_idx_hbm.at[pl.ds(ibase, b_per_w)], tok_idx_v)
        pltpu.sync_copy(pos_idx_hbm.at[pl.ds(ibase, b_per_w)], pos_idx_v)

        lane = lax.broadcasted_iota(jnp.int32, (16,), 0)
        dt_vecs = [(seg * 16 + lane) // 8 for seg in range(D // 16)]
        dl_vecs = [lane % 8] * (D // 16)

        def run_table(tab, idx_v, out):
            def gdesc(k, b):
                start = pl.multiple_of(k * W, 8)
                return pltpu.make_async_copy(
                    tab.at[idx_v.at[pl.ds(start, W)]], rows[b], gsems[b])

            def odesc(k, b):
                g = gbase + k
                bb = g // NA
                at = g % NA
                return pltpu.make_async_copy(
                    rowsT[b], out.at[bb, :, at, :, :], osems[b])

            def transpose(b):
                src = rows[b]
                dst = rowsT[b]

                def tbody(j, carry):
                    al0 = j * TUNROLL
                    for i in range(TUNROLL):
                        al16 = jnp.full((16,), al0 + i, dtype=jnp.int32)
                        for seg in range(D // 16):
                            vec = src[al0 + i, pl.ds(seg * 16, 16)]
                            plsc.store_scatter(
                                dst, [dt_vecs[seg], dl_vecs[seg], al16], vec)
                    return carry

                lax.fori_loop(0, W // TUNROLL, tbody, 0)

            for b in range(NBUF):
                gdesc(b, b).start()

            def body(j, carry):
                for b in range(NBUF):
                    k = j * NBUF + b
                    gdesc(k, b).wait()
                    transpose(b)
                    odesc(k, b).start()
                for b in range(NBUF):
                    k = j * NBUF + b
                    odesc(k, b).wait()
                    gdesc(k + NBUF, b).start()
                return carry

            lax.fori_loop(0, ngroup - 1, body, 0)

            last = (ngroup - 1) * NBUF
            for b in range(NBUF):
                gdesc(last + b, b).wait()
                transpose(b)
                odesc(last + b, b).start()
            for b in range(NBUF):
                odesc(last + b, b).wait()

        run_table(tok_tab, tok_idx_v, tok_out)
        run_table(pos_tab, pos_idx_v, pos_out)

    return lookup


def kernel(tokens, pos, token_table, pos_table):
    S0, S1 = tokens.shape
    B = S0 * S1
    D = token_table.shape[1]
    tok_flat = tokens.T.reshape(B).astype(jnp.int32)
    pos_flat = pos.T.reshape(B).astype(jnp.int32)
    tok5, pos5 = _make_lookup(S0, S1, D)(
        tok_flat, pos_flat, token_table, pos_table)

    def to_entry(o5):
        return o5.transpose(2, 4, 0, 1, 3).reshape(S0, S1, D)

    return to_entry(tok5), to_entry(pos5)


# speedup vs baseline: 1.2608x; 1.2608x over previous
import functools

import jax
import jax.numpy as jnp
from jax import lax
from jax.experimental import pallas as pl
from jax.experimental.pallas import tpu as pltpu
from jax.experimental.pallas import tpu_sc as plsc

NC = 2
NS = 16
NW = NC * NS
W = 128
NBUF = 4
TUNROLL = 8


@functools.lru_cache(maxsize=None)
def _make_lookup(S0, S1, D):
    B = S0 * S1
    NA = S0 // W
    b_per_w = B // NW
    nblk = b_per_w // W
    ngroup = nblk // NBUF
    assert b_per_w * NW == B and W * nblk == b_per_w and NBUF * ngroup == nblk
    assert D == 64 and S0 % W == 0

    mesh = plsc.VectorSubcoreMesh(core_axis_name="c", subcore_axis_name="s")

    @functools.partial(
        pl.kernel,
        mesh=mesh,
        compiler_params=pltpu.CompilerParams(
            use_tc_tiling_on_sc=False, needs_layout_passes=False),
        out_type=(
            jax.ShapeDtypeStruct((S1, 8, NA, 8, W), jnp.float32),
            jax.ShapeDtypeStruct((S1, 8, NA, 8, W), jnp.float32),
        ),
        scratch_types=(
            [pltpu.VMEM((b_per_w,), jnp.int32)] * 2
            + [pltpu.VMEM((W, D), jnp.float32)] * NBUF
            + [pltpu.VMEM((8, 8, W), jnp.float32)] * NBUF
            + [pltpu.SemaphoreType.DMA] * (2 * NBUF)
        ),
    )
    def lookup(tok_idx_hbm, pos_idx_hbm, tok_tab, pos_tab, tok_out, pos_out,
               tok_idx_v, pos_idx_v, *scratch):
        rows = scratch[:NBUF]
        rowsT = scratch[NBUF:2 * NBUF]
        gsems = scratch[2 * NBUF:3 * NBUF]
        osems = scratch[3 * NBUF:]

        wid = lax.axis_index("s") * NC + lax.axis_index("c")
        ibase = pl.multiple_of(wid * b_per_w, 8)
        gbase = wid * nblk

        pltpu.sync_copy(tok_idx_hbm.at[pl.ds(ibase, b_per_w)], tok_idx_v)
        pltpu.sync_copy(pos_idx_hbm.at[pl.ds(ibase, b_per_w)], pos_idx_v)

        lane = lax.broadcasted_iota(jnp.int32, (16,), 0)
        dt_vecs = [(seg * 16 + lane) // 8 for seg in range(D // 16)]
        dl_vecs = [lane % 8] * (D // 16)

        def run_table(tab, idx_v, out):
            def gdesc(k, b):
                start = pl.multiple_of(k * W, 8)
                return pltpu.make_async_copy(
                    tab.at[idx_v.at[pl.ds(start, W)]], rows[b], gsems[b])

            def odesc(k, b):
                g = gbase + k
                bb = g // NA
                at = g % NA
                return pltpu.make_async_copy(
                    rowsT[b], out.at[bb, :, at, :, :], osems[b])

            def transpose(b):
                src = rows[b]
                dst = rowsT[b]

                def tbody(a16, carry):
                    al0 = a16 * 16
                    rvec = al0 + lane
                    for dc in range(D // 16):
                        vecs = [
                            plsc.load_gather(
                                src,
                                [rvec,
                                 jnp.full((16,), dc * 16 + i, jnp.int32)])
                            for i in range(16)
                        ]
                        for i, v in enumerate(vecs):
                            d = dc * 16 + i
                            dst[d // 8, d % 8, pl.ds(al0, 16)] = v
                    return carry

                lax.fori_loop(0, W // 16, tbody, 0)

            for b in range(NBUF):
                gdesc(b, b).start()

            def body(j, carry):
                for b in range(NBUF):
                    k = j * NBUF + b
                    gdesc(k, b).wait()
                    transpose(b)
                    odesc(k, b).start()
                for b in range(NBUF):
                    k = j * NBUF + b
                    odesc(k, b).wait()

                    @pl.when(j < ngroup - 1)
                    def _():
                        gdesc(k + NBUF, b).start()
                return carry

            lax.fori_loop(0, ngroup, body, 0)

        run_table(tok_tab, tok_idx_v, tok_out)
        run_table(pos_tab, pos_idx_v, pos_out)

    return lookup


def kernel(tokens, pos, token_table, pos_table):
    S0, S1 = tokens.shape
    B = S0 * S1
    D = token_table.shape[1]
    tok_flat = tokens.T.reshape(B).astype(jnp.int32)
    pos_flat = pos.T.reshape(B).astype(jnp.int32)
    tok5, pos5 = _make_lookup(S0, S1, D)(
        tok_flat, pos_flat, token_table, pos_table)

    def to_entry(o5):
        return o5.transpose(2, 4, 0, 1, 3).reshape(S0, S1, D)

    return to_entry(tok5), to_entry(pos5)


# speedup vs baseline: 2.0530x; 1.6283x over previous
import functools

import jax
import jax.numpy as jnp
from jax import lax
from jax.experimental import pallas as pl
from jax.experimental.pallas import tpu as pltpu
from jax.experimental.pallas import tpu_sc as plsc

NC = 2
NS = 16
NW = NC * NS
W = 128
NBUF = 4
TUNROLL = 8


@functools.lru_cache(maxsize=None)
def _make_lookup(S0, S1, D):
    B = S0 * S1
    NA = S0 // W
    b_per_w = B // NW
    nblk = b_per_w // W
    ngroup = nblk // NBUF
    assert b_per_w * NW == B and W * nblk == b_per_w and NBUF * ngroup == nblk
    assert D == 64 and S0 % W == 0

    mesh = plsc.VectorSubcoreMesh(core_axis_name="c", subcore_axis_name="s")

    @functools.partial(
        pl.kernel,
        mesh=mesh,
        compiler_params=pltpu.CompilerParams(
            use_tc_tiling_on_sc=False, needs_layout_passes=False),
        out_type=(
            jax.ShapeDtypeStruct((S1, 8, NA, 8, W), jnp.float32),
            jax.ShapeDtypeStruct((S1, 8, NA, 8, W), jnp.float32),
        ),
        scratch_types=(
            [pltpu.VMEM((b_per_w,), jnp.int32)] * 2
            + [pltpu.VMEM((W, D), jnp.float32)] * NBUF
            + [pltpu.VMEM((8, 8, W + 1), jnp.float32)] * NBUF
            + [pltpu.SemaphoreType.DMA] * (2 * NBUF)
        ),
    )
    def lookup(tok_idx_hbm, pos_idx_hbm, tok_tab, pos_tab, tok_out, pos_out,
               tok_idx_v, pos_idx_v, *scratch):
        rows = scratch[:NBUF]
        rowsT = scratch[NBUF:2 * NBUF]
        gsems = scratch[2 * NBUF:3 * NBUF]
        osems = scratch[3 * NBUF:]

        wid = lax.axis_index("s") * NC + lax.axis_index("c")
        ibase = pl.multiple_of(wid * b_per_w, 8)
        gbase = wid * nblk

        pltpu.sync_copy(tok_idx_hbm.at[pl.ds(ibase, b_per_w)], tok_idx_v)
        pltpu.sync_copy(pos_idx_hbm.at[pl.ds(ibase, b_per_w)], pos_idx_v)

        lane = lax.broadcasted_iota(jnp.int32, (16,), 0)
        dt_vecs = [(seg * 16 + lane) // 8 for seg in range(D // 16)]
        dl_vecs = [lane % 8] * (D // 16)

        def run_table(tab, idx_v, out):
            def gdesc(k, b):
                start = pl.multiple_of(k * W, 8)
                return pltpu.make_async_copy(
                    tab.at[idx_v.at[pl.ds(start, W)]], rows[b], gsems[b])

            def odesc(k, b):
                g = gbase + k
                bb = g // NA
                at = g % NA
                return pltpu.make_async_copy(
                    rowsT[b].at[:, :, pl.ds(0, W)], out.at[bb, :, at, :, :],
                    osems[b])

            def transpose(b):
                src = rows[b]
                dst = rowsT[b]

                def tbody(j, carry):
                    al0 = j * TUNROLL
                    for i in range(TUNROLL):
                        al16 = jnp.full((16,), al0 + i, dtype=jnp.int32)
                        vecs = [src[al0 + i, pl.ds(seg * 16, 16)]
                                for seg in range(D // 16)]
                        for seg in range(D // 16):
                            plsc.store_scatter(
                                dst, [dt_vecs[seg], dl_vecs[seg], al16],
                                vecs[seg])
                    return carry

                lax.fori_loop(0, W // TUNROLL, tbody, 0)

            for b in range(NBUF):
                gdesc(b, b).start()

            def body(j, carry):
                for b in range(NBUF):
                    k = j * NBUF + b
                    gdesc(k, b).wait()
                    transpose(b)
                    odesc(k, b).start()
                for b in range(NBUF):
                    k = j * NBUF + b
                    odesc(k, b).wait()

                    @pl.when(j < ngroup - 1)
                    def _():
                        gdesc(k + NBUF, b).start()
                return carry

            lax.fori_loop(0, ngroup, body, 0)

        run_table(tok_tab, tok_idx_v, tok_out)
        run_table(pos_tab, pos_idx_v, pos_out)

    return lookup


def kernel(tokens, pos, token_table, pos_table):
    S0, S1 = tokens.shape
    B = S0 * S1
    D = token_table.shape[1]
    tok_flat = tokens.T.reshape(B).astype(jnp.int32)
    pos_flat = pos.T.reshape(B).astype(jnp.int32)
    tok5, pos5 = _make_lookup(S0, S1, D)(
        tok_flat, pos_flat, token_table, pos_table)

    def to_entry(o5):
        return o5.transpose(2, 4, 0, 1, 3).reshape(S0, S1, D)

    return to_entry(tok5), to_entry(pos5)
